# baseline (speedup 1.0000x reference)
import functools

import jax
import jax.numpy as jnp
from jax import lax
from jax.experimental import pallas as pl
from jax.experimental.pallas import tpu as pltpu
from jax.experimental.pallas import tpu_sc as plsc

_NC = 2
_NS = 16
_NW = _NC * _NS
_CHUNK = 128
_NPIPE = 4
_TM = 1024


@functools.lru_cache(maxsize=None)
def _make_gather(ntok: int, d: int):
    b_per_w = ntok // _NW
    nchunk = b_per_w // _CHUNK
    mesh = plsc.VectorSubcoreMesh(core_axis_name="c", subcore_axis_name="s")

    @functools.partial(
        pl.kernel,
        out_type=jax.ShapeDtypeStruct((ntok, d), jnp.float32),
        mesh=mesh,
        scratch_types=[
            pltpu.VMEM((nchunk, _CHUNK), jnp.int32),
            pltpu.VMEM((b_per_w, d), jnp.float32),
            pltpu.SemaphoreType.DMA,
        ],
    )
    def gather_kernel(idx_hbm, table_hbm, out_hbm, idx_v, rows_v, sem):
        wid = lax.axis_index("s") * _NC + lax.axis_index("c")
        base = wid * b_per_w
        pltpu.sync_copy(idx_hbm.at[wid], idx_v)
        copies = []
        for j in range(nchunk):
            copies.append(
                pltpu.async_copy(
                    table_hbm.at[idx_v.at[j]],
                    rows_v.at[pl.ds(j * _CHUNK, _CHUNK)],
                    sem,
                )
            )
        for c in copies:
            c.wait()
        pltpu.sync_copy(rows_v, out_hbm.at[pl.ds(base, b_per_w)])

    return gather_kernel


def _proj_first_body(x_ref, w_ref, s_ref, o_ref):
    o_ref[...] = (
        lax.dot_general(
            x_ref[...],
            w_ref[...],
            (((1,), (1,)), ((), ())),
            preferred_element_type=jnp.float32,
        )
        * s_ref[0]
    )


def _proj_chain_body(acc_ref, x_ref, w_ref, s_ref, o_ref):
    del acc_ref
    o_ref[...] = (
        lax.dot_general(
            x_ref[...],
            w_ref[...],
            (((1,), (1,)), ((), ())),
            preferred_element_type=jnp.float32,
        )
        * s_ref[0]
    )


@functools.lru_cache(maxsize=None)
def _make_proj(ntok: int, d: int, m: int, rows: int, row_base: int, first: bool):
    grid = (rows // _TM,)
    base_blk = row_base // _TM
    x_spec = pl.BlockSpec((_TM, d), lambda i: (i, 0))
    w_spec = pl.BlockSpec((m, d), lambda i: (0, 0))
    s_spec = pl.BlockSpec(memory_space=pltpu.SMEM)
    out_spec = pl.BlockSpec((_TM, m), lambda i: (base_blk + i, 0))
    out_shape = jax.ShapeDtypeStruct((ntok, m), jnp.float32)
    if first:
        return pl.pallas_call(
            _proj_first_body,
            grid=grid,
            in_specs=[x_spec, w_spec, s_spec],
            out_specs=out_spec,
            out_shape=out_shape,
        )
    return pl.pallas_call(
        _proj_chain_body,
        grid=grid,
        in_specs=[
            pl.BlockSpec(memory_space=pl.ANY),
            x_spec,
            w_spec,
            s_spec,
        ],
        out_specs=out_spec,
        out_shape=out_shape,
        input_output_aliases={0: 0},
    )


def kernel(token_ids, embed_table, proj_weight, scale):
    b, s = token_ids.shape
    ntok = b * s
    d = embed_table.shape[1]
    m = proj_weight.shape[0]
    scale1 = scale.astype(jnp.float32).reshape(1)
    rows = ntok // _NPIPE
    ids = token_ids.astype(jnp.int32).reshape(
        _NPIPE, _NW, rows // _NW // _CHUNK, _CHUNK
    )
    gather = _make_gather(rows, d)
    gathered = [gather(ids[k], embed_table) for k in range(_NPIPE)]
    out = _make_proj(ntok, d, m, rows, 0, True)(gathered[0], proj_weight, scale1)
    for k in range(1, _NPIPE):
        out = _make_proj(ntok, d, m, rows, k * rows, False)(
            out, gathered[k], proj_weight, scale1
        )
    return out.reshape(b, s, m)


# speedup vs baseline: 1.0745x; 1.0745x over previous
import functools

import jax
import jax.numpy as jnp
from jax import lax
from jax.experimental import pallas as pl
from jax.experimental.pallas import tpu as pltpu
from jax.experimental.pallas import tpu_sc as plsc

_NC = 2
_NS = 16
_NW = _NC * _NS
_CHUNK = 128
_TM = 1024


@functools.lru_cache(maxsize=None)
def _make_gather(ntok: int, d: int):
    b_per_w = ntok // _NW
    nchunk = b_per_w // _CHUNK
    mesh = plsc.VectorSubcoreMesh(core_axis_name="c", subcore_axis_name="s")

    @functools.partial(
        pl.kernel,
        out_type=jax.ShapeDtypeStruct((ntok, d), jnp.float32),
        mesh=mesh,
        scratch_types=[
            pltpu.VMEM((nchunk, _CHUNK), jnp.int32),
            pltpu.VMEM((b_per_w, d), jnp.float32),
            pltpu.SemaphoreType.DMA,
        ],
    )
    def gather_kernel(idx_hbm, table_hbm, out_hbm, idx_v, rows_v, sem):
        wid = lax.axis_index("s") * _NC + lax.axis_index("c")
        base = wid * b_per_w
        pltpu.sync_copy(idx_hbm.at[wid], idx_v)
        copies = []
        for j in range(nchunk):
            copies.append(
                pltpu.async_copy(
                    table_hbm.at[idx_v.at[j]],
                    rows_v.at[pl.ds(j * _CHUNK, _CHUNK)],
                    sem,
                )
            )
        for c in copies:
            c.wait()
        pltpu.sync_copy(rows_v, out_hbm.at[pl.ds(base, b_per_w)])

    return gather_kernel


def _proj_body(x_ref, w_ref, s_ref, o_ref):
    o_ref[...] = (
        lax.dot_general(
            x_ref[...],
            w_ref[...],
            (((1,), (1,)), ((), ())),
            preferred_element_type=jnp.float32,
        )
        * s_ref[0]
    )


@functools.lru_cache(maxsize=None)
def _make_proj(ntok: int, d: int, m: int):
    grid = (ntok // _TM,)
    return pl.pallas_call(
        _proj_body,
        grid=grid,
        in_specs=[
            pl.BlockSpec((_TM, d), lambda i: (i, 0)),
            pl.BlockSpec((m, d), lambda i: (0, 0)),
            pl.BlockSpec(memory_space=pltpu.SMEM),
        ],
        out_specs=pl.BlockSpec((_TM, m), lambda i: (i, 0)),
        out_shape=jax.ShapeDtypeStruct((ntok, m), jnp.float32),
    )


@functools.lru_cache(maxsize=None)
def _make_proj_slice(ntok: int, d: int, m: int, rows: int, row_base: int,
                     aliased: bool):
    grid = (rows // _TM,)
    base_blk = row_base // _TM
    x_spec = pl.BlockSpec((_TM, d), lambda i: (i, 0))
    w_spec = pl.BlockSpec((m, d), lambda i: (0, 0))
    s_spec = pl.BlockSpec(memory_space=pltpu.SMEM)
    out_spec = pl.BlockSpec((_TM, m), lambda i: (base_blk + i, 0))
    out_shape = jax.ShapeDtypeStruct((ntok, m), jnp.float32)
    if not aliased:
        return pl.pallas_call(
            _proj_body,
            grid=grid,
            in_specs=[x_spec, w_spec, s_spec],
            out_specs=out_spec,
            out_shape=out_shape,
        )

    def body(acc_ref, x_ref, w_ref, s_ref, o_ref):
        del acc_ref
        _proj_body(x_ref, w_ref, s_ref, o_ref)

    return pl.pallas_call(
        body,
        grid=grid,
        in_specs=[pl.BlockSpec(memory_space=pl.ANY), x_spec, w_spec, s_spec],
        out_specs=out_spec,
        out_shape=out_shape,
        input_output_aliases={0: 0},
    )


_NPIPE = 2


def kernel(token_ids, embed_table, proj_weight, scale):
    b, s = token_ids.shape
    ntok = b * s
    d = embed_table.shape[1]
    m = proj_weight.shape[0]
    scale1 = scale.astype(jnp.float32).reshape(1)
    rows = ntok // _NPIPE
    ids = token_ids.astype(jnp.int32).reshape(
        _NPIPE, _NW, rows // _NW // _CHUNK, _CHUNK
    )
    gather = _make_gather(rows, d)
    gathered = [gather(ids[k], embed_table) for k in range(_NPIPE)]
    out = _make_proj_slice(ntok, d, m, rows, 0, False)(
        gathered[0], proj_weight, scale1
    )
    for k in range(1, _NPIPE):
        out = _make_proj_slice(ntok, d, m, rows, k * rows, True)(
            out, gathered[k], proj_weight, scale1
        )
    return out.reshape(b, s, m)


# speedup vs baseline: 1.1266x; 1.0485x over previous
import functools

import jax
import jax.numpy as jnp
from jax import lax
from jax.experimental import pallas as pl
from jax.experimental.pallas import tpu as pltpu
from jax.experimental.pallas import tpu_sc as plsc

_NC = 2
_NS = 16
_NW = _NC * _NS
_CHUNK = 128
_TM = 1024


@functools.lru_cache(maxsize=None)
def _make_gather(ntok: int, d: int):
    b_per_w = ntok // _NW
    nchunk = b_per_w // _CHUNK
    mesh = plsc.VectorSubcoreMesh(core_axis_name="c", subcore_axis_name="s")

    @functools.partial(
        pl.kernel,
        out_type=jax.ShapeDtypeStruct((ntok, d), jnp.float32),
        mesh=mesh,
        scratch_types=[
            pltpu.VMEM((nchunk, _CHUNK), jnp.int32),
            pltpu.VMEM((b_per_w, d), jnp.float32),
            pltpu.SemaphoreType.DMA,
        ],
    )
    def gather_kernel(idx_hbm, table_hbm, out_hbm, idx_v, rows_v, sem):
        wid = lax.axis_index("s") * _NC + lax.axis_index("c")
        base = wid * b_per_w
        pltpu.sync_copy(idx_hbm.at[wid], idx_v)
        copies = []
        for j in range(nchunk):
            copies.append(
                pltpu.async_copy(
                    table_hbm.at[idx_v.at[j]],
                    rows_v.at[pl.ds(j * _CHUNK, _CHUNK)],
                    sem,
                )
            )
        for c in copies:
            c.wait()
        pltpu.sync_copy(rows_v, out_hbm.at[pl.ds(base, b_per_w)])

    return gather_kernel


def _proj_body(x_ref, w_ref, s_ref, o_ref):
    o_ref[...] = (
        lax.dot_general(
            x_ref[...],
            w_ref[...],
            (((1,), (1,)), ((), ())),
            preferred_element_type=jnp.float32,
        )
        * s_ref[0]
    )


@functools.lru_cache(maxsize=None)
def _make_proj(ntok: int, d: int, m: int):
    grid = (ntok // _TM,)
    return pl.pallas_call(
        _proj_body,
        grid=grid,
        in_specs=[
            pl.BlockSpec((_TM, d), lambda i: (i, 0)),
            pl.BlockSpec((m, d), lambda i: (0, 0)),
            pl.BlockSpec(memory_space=pltpu.SMEM),
        ],
        out_specs=pl.BlockSpec((_TM, m), lambda i: (i, 0)),
        out_shape=jax.ShapeDtypeStruct((ntok, m), jnp.float32),
    )


@functools.lru_cache(maxsize=None)
def _make_proj_slice(ntok: int, d: int, m: int, rows: int, row_base: int,
                     aliased: bool):
    grid = (rows // _TM,)
    base_blk = row_base // _TM
    x_spec = pl.BlockSpec((_TM, d), lambda i: (i, 0))
    w_spec = pl.BlockSpec((m, d), lambda i: (0, 0))
    s_spec = pl.BlockSpec(memory_space=pltpu.SMEM)
    out_spec = pl.BlockSpec((_TM, m), lambda i: (base_blk + i, 0))
    out_shape = jax.ShapeDtypeStruct((ntok, m), jnp.float32)
    if not aliased:
        return pl.pallas_call(
            _proj_body,
            grid=grid,
            in_specs=[x_spec, w_spec, s_spec],
            out_specs=out_spec,
            out_shape=out_shape,
        )

    def body(acc_ref, x_ref, w_ref, s_ref, o_ref):
        del acc_ref
        _proj_body(x_ref, w_ref, s_ref, o_ref)

    return pl.pallas_call(
        body,
        grid=grid,
        in_specs=[pl.BlockSpec(memory_space=pl.ANY), x_spec, w_spec, s_spec],
        out_specs=out_spec,
        out_shape=out_shape,
        input_output_aliases={0: 0},
    )


_NPIPE = 1


def kernel(token_ids, embed_table, proj_weight, scale):
    b, s = token_ids.shape
    ntok = b * s
    d = embed_table.shape[1]
    m = proj_weight.shape[0]
    scale1 = scale.astype(jnp.float32).reshape(1)
    rows = ntok // _NPIPE
    ids = token_ids.astype(jnp.int32).reshape(
        _NPIPE, _NW, rows // _NW // _CHUNK, _CHUNK
    )
    gather = _make_gather(rows, d)
    gathered = [gather(ids[k], embed_table) for k in range(_NPIPE)]
    out = _make_proj_slice(ntok, d, m, rows, 0, False)(
        gathered[0], proj_weight, scale1
    )
    for k in range(1, _NPIPE):
        out = _make_proj_slice(ntok, d, m, rows, k * rows, True)(
            out, gathered[k], proj_weight, scale1
        )
    return out.reshape(b, s, m)
